# Initial kernel scaffold; baseline (speedup 1.0000x reference)
import jax
import jax.numpy as jnp
from jax.experimental import pallas as pl


def kernel(h, e, edge_index, A_w, A_b, B_w, B_b, C_w, C_b, D_w, D_b, E_w, E_b, bn_h_w, bn_h_b, bn_e_w, bn_e_b):
    raise NotImplementedError("write your pallas kernel here")


# speedup vs baseline: 1.3660x; 1.3660x over previous
import functools

import jax
import jax.numpy as jnp
from jax import lax
from jax.experimental import pallas as pl
from jax.experimental.pallas import tpu as pltpu
from jax.experimental.pallas import tpu_sc as plsc

NC = 2
NS = 16
LANES = 16

K = 80
ZROWS = K


def _half_store(ref, x, d_half):
    ref[0] = x[:, :d_half]
    ref[1] = x[:, d_half:]


def _node_linear(h, wa, ba, wb, bb, wd, bd, we, be):
    n, d = h.shape
    dh_ = d // 2

    def body(h_ref, wa_ref, ba_ref, wb_ref, bb_ref, wd_ref, bd_ref,
             we_ref, be_ref, ah_ref, p_ref, eh_ref):
        hv = h_ref[...]

        def mm(w_ref, b_ref):
            return jnp.dot(hv, w_ref[...],
                           preferred_element_type=jnp.float32) + b_ref[...]

        ah_ref[...] = mm(wa_ref, ba_ref)
        bh = mm(wb_ref, bb_ref)
        dh = mm(wd_ref, bd_ref)
        p_ref[0] = jnp.concatenate([dh[:, :dh_], bh[:, :dh_]], axis=1)
        p_ref[1] = jnp.concatenate([dh[:, dh_:], bh[:, dh_:]], axis=1)
        eh_ref[...] = mm(we_ref, be_ref)

    return pl.pallas_call(
        body,
        out_shape=[jax.ShapeDtypeStruct((n, d), jnp.float32),
                   jax.ShapeDtypeStruct((2, n, d), jnp.float32),
                   jax.ShapeDtypeStruct((n, d), jnp.float32)],
    )(h, wa, ba, wb, bb, wd, bd, we, be)


def _edge_linear(e, wc, bc, block_e):
    ee, d = e.shape
    dh_ = d // 2

    def body(e_ref, w_ref, b_ref, out_ref):
        x = jnp.dot(e_ref[...], w_ref[...],
                    preferred_element_type=jnp.float32) + b_ref[...]
        _half_store(out_ref, x, dh_)

    return pl.pallas_call(
        body,
        grid=(ee // block_e,),
        in_specs=[
            pl.BlockSpec((block_e, d), lambda i: (i, 0)),
            pl.BlockSpec((d, d), lambda i: (0, 0)),
            pl.BlockSpec((1, d), lambda i: (0, 0)),
        ],
        out_specs=pl.BlockSpec((2, block_e, dh_), lambda i: (0, i, 0)),
        out_shape=jax.ShapeDtypeStruct((2, ee, dh_), jnp.float32),
    )(e, wc, bc)


def _sc_edge_pass(ce_s, p_s, eh_f, src, dst, n, ee):
    dh_ = ce_s.shape[-1]
    d = 2 * dh_
    epw = ee // NS
    nch = epw // K
    nzt = n // ZROWS
    nzc = (nzt + NS - 1) // NS
    ngr = dh_ // LANES

    mesh = plsc.VectorSubcoreMesh(core_axis_name="c", subcore_axis_name="s")

    @functools.partial(
        pl.kernel,
        mesh=mesh,
        out_type=[
            jax.ShapeDtypeStruct((2, ee, dh_), jnp.float32),
            jax.ShapeDtypeStruct((2, n, d), jnp.float32),
            jax.ShapeDtypeStruct((2, NS, 8, dh_), jnp.float32),
            jax.ShapeDtypeStruct((2, NS, 8, dh_), jnp.float32),
        ],
        scratch_types=[
            pltpu.VMEM((K,), jnp.int32),
            pltpu.VMEM((K,), jnp.int32),
            pltpu.VMEM((K, dh_), jnp.float32),
            pltpu.VMEM((K, d), jnp.float32),
            pltpu.VMEM((K, d), jnp.float32),
            pltpu.VMEM((K, d), jnp.float32),
            pltpu.VMEM((16, dh_), jnp.float32),
            pltpu.VMEM_SHARED((n, d), jnp.float32),
            pltpu.SemaphoreType.DMA,
        ],
    )
    def sc_kernel(ce_hbm, p_hbm, eh_hbm, src_hbm, dst_hbm,
                  eij_hbm, nd_hbm, ssum_hbm, ssq_hbm,
                  srcv, dstv, cebuf, pbuf, ehbuf, cbuf, statbuf,
                  acc_sh, sem):
        c = lax.axis_index("c")
        s = lax.axis_index("s")
        zero = jnp.zeros((LANES,), jnp.float32)

        @pl.loop(0, ZROWS)
        def _zrow(i):
            for j in range(2 * ngr):
                cbuf[i, pl.ds(j * LANES, LANES)] = zero

        @pl.loop(0, 16)
        def _zstat(i):
            for j in range(ngr):
                statbuf[i, pl.ds(j * LANES, LANES)] = zero

        @pl.loop(0, nzc)
        def _zchunk(k):
            ci = s + k * NS

            @pl.when(ci < nzt)
            def _():
                pltpu.sync_copy(cbuf, acc_sh.at[pl.ds(ci * ZROWS, ZROWS)])

        plsc.subcore_barrier()

        @pl.loop(0, nch)
        def _chunk(t):
            base = s * epw + t * K
            pltpu.sync_copy(src_hbm.at[pl.ds(base, K)], srcv)
            pltpu.sync_copy(dst_hbm.at[pl.ds(base, K)], dstv)
            pltpu.sync_copy(ce_hbm.at[c].at[pl.ds(base, K)], cebuf)
            pltpu.sync_copy(p_hbm.at[c].at[srcv], pbuf)
            pltpu.sync_copy(eh_hbm.at[dstv], ehbuf)

            def row(i, carry):
                acc = list(carry)
                for j in range(ngr):
                    slc = pl.ds(j * LANES, LANES)
                    hslc = pl.ds(dh_ + j * LANES, LANES)
                    x = (cebuf[i, slc] + pbuf[i, slc]
                         + ehbuf[i, pl.ds(c * dh_ + j * LANES, LANES)])
                    cebuf[i, slc] = x
                    sg = 1.0 / (1.0 + jnp.exp(-x))
                    cbuf[i, hslc] = sg
                    cbuf[i, slc] = sg * pbuf[i, hslc]
                    acc[j] = acc[j] + x
                    acc[ngr + j] = acc[ngr + j] + x * x
                return tuple(acc)

            init = tuple(statbuf[0, pl.ds(j * LANES, LANES)] for j in range(ngr)) \
                 + tuple(statbuf[8, pl.ds(j * LANES, LANES)] for j in range(ngr))
            fin = lax.fori_loop(0, K, row, init)
            for j in range(ngr):
                statbuf[0, pl.ds(j * LANES, LANES)] = fin[j]
                statbuf[8, pl.ds(j * LANES, LANES)] = fin[ngr + j]

            pltpu.sync_copy(cebuf, eij_hbm.at[c].at[pl.ds(base, K)])
            pltpu.sync_copy(cbuf, acc_sh.at[dstv], add=True)

        pltpu.sync_copy(statbuf.at[pl.ds(0, 8)], ssum_hbm.at[c].at[s])
        pltpu.sync_copy(statbuf.at[pl.ds(8, 8)], ssq_hbm.at[c].at[s])

        plsc.subcore_barrier()

        @pl.loop(0, nzc)
        def _dump(k):
            ci = s + k * NS

            @pl.when(ci < nzt)
            def _():
                r0 = ci * ZROWS
                pltpu.sync_copy(acc_sh.at[pl.ds(r0, ZROWS)],
                                nd_hbm.at[c].at[pl.ds(r0, ZROWS)])

    return sc_kernel(ce_s, p_s, eh_f, src, dst)


def _finalize_h(ah, nd_s, ssum, ssq, bnhw, bnhb, bnew, bneb, ee):
    n, d = ah.shape
    dh_ = d // 2

    def body(ah_ref, nd_ref, ssum_ref, ssq_ref,
             bnhw_ref, bnhb_ref, bnew_ref, bneb_ref, hout_ref, ess_ref):
        num = jnp.concatenate([nd_ref[0][:, :dh_], nd_ref[1][:, :dh_]], axis=1)
        den = jnp.concatenate([nd_ref[0][:, dh_:], nd_ref[1][:, dh_:]], axis=1)
        hpre = ah_ref[...] + num / (den + 1e-6)
        mean = jnp.mean(hpre, axis=0, keepdims=True)
        var = jnp.mean((hpre - mean) ** 2, axis=0, keepdims=True)
        hn = (hpre - mean) * lax.rsqrt(var + 1e-5) * bnhw_ref[...] + bnhb_ref[...]
        hout_ref[...] = jnp.maximum(hn, 0.0)

        def red(ref, i):
            x = ref[i]
            return jnp.sum(x.reshape(x.shape[0] * x.shape[1], x.shape[2]),
                           axis=0, keepdims=True)

        esum = jnp.concatenate([red(ssum_ref, 0), red(ssum_ref, 1)], axis=1)
        esq = jnp.concatenate([red(ssq_ref, 0), red(ssq_ref, 1)], axis=1)
        inv_e = 1.0 / float(ee)
        emean = esum * inv_e
        evar = esq * inv_e - emean * emean
        escale = bnew_ref[...] * lax.rsqrt(evar + 1e-5)
        eshift = bneb_ref[...] - emean * escale
        ess_ref[0:1] = escale
        ess_ref[1:2] = eshift

    return pl.pallas_call(
        body,
        out_shape=[jax.ShapeDtypeStruct((n, d), jnp.float32),
                   jax.ShapeDtypeStruct((2, d), jnp.float32)],
    )(ah, nd_s, ssum, ssq, bnhw, bnhb, bnew, bneb)


def _apply_e(eij_s, ess, block_e):
    _, ee, dh_ = eij_s.shape
    d = 2 * dh_

    def body(eij_ref, ess_ref, out_ref):
        x = jnp.concatenate([eij_ref[0], eij_ref[1]], axis=1)
        y = x * ess_ref[0:1] + ess_ref[1:2]
        out_ref[...] = jnp.maximum(y, 0.0)

    return pl.pallas_call(
        body,
        grid=(ee // block_e,),
        in_specs=[
            pl.BlockSpec((2, block_e, dh_), lambda i: (0, i, 0)),
            pl.BlockSpec((2, d), lambda i: (0, 0)),
        ],
        out_specs=pl.BlockSpec((block_e, d), lambda i: (i, 0)),
        out_shape=jax.ShapeDtypeStruct((ee, d), jnp.float32),
    )(eij_s, ess)


def kernel(h, e, edge_index, A_w, A_b, B_w, B_b, C_w, C_b, D_w, D_b,
           E_w, E_b, bn_h_w, bn_h_b, bn_e_w, bn_e_b):
    n, d = h.shape
    ee = e.shape[0]
    src = edge_index[0]
    dst = edge_index[1]
    row = lambda v: v.reshape(1, -1)

    ah, p_s, eh_f = _node_linear(
        h, A_w.T, row(A_b), B_w.T, row(B_b), D_w.T, row(D_b), E_w.T, row(E_b))
    ce_s = _edge_linear(e, C_w.T, row(C_b), block_e=4000)
    eij_s, nd_s, ssum, ssq = _sc_edge_pass(
        ce_s, p_s, eh_f, src, dst, n, ee)
    h_out, ess = _finalize_h(
        ah, nd_s, ssum, ssq,
        row(bn_h_w), row(bn_h_b), row(bn_e_w), row(bn_e_b), ee)
    e_out = _apply_e(eij_s, ess, block_e=4000)
    return (h_out, e_out)
